# Initial kernel scaffold; baseline (speedup 1.0000x reference)
import jax
import jax.numpy as jnp
from jax.experimental import pallas as pl


def kernel(x, W_E):
    raise NotImplementedError("write your pallas kernel here")


# speedup vs baseline: 2.2830x; 2.2830x over previous
import functools

import jax
import jax.numpy as jnp
from jax import lax
from jax.experimental import pallas as pl
from jax.experimental.pallas import tpu as pltpu
from jax.experimental.pallas import tpu_sc as plsc

D_MODEL = 128
VOCAB = 100000
_VB = 2048

_NC = 2
_NS = 16
_NW = _NC * _NS
_CHUNK = 128


def _transpose_body(w_ref, o_ref):
    o_ref[...] = w_ref[...].T


def _transpose(W_E):
    return pl.pallas_call(
        _transpose_body,
        grid=((VOCAB + _VB - 1) // _VB,),
        in_specs=[pl.BlockSpec((D_MODEL, _VB), lambda i: (0, i))],
        out_specs=pl.BlockSpec((_VB, D_MODEL), lambda i: (i, 0)),
        out_shape=jax.ShapeDtypeStruct((VOCAB, D_MODEL), jnp.float32),
    )(W_E)


def _gather(table_t, idx3d, n_tokens):
    per_w = idx3d.shape[1]
    mesh = plsc.VectorSubcoreMesh(core_axis_name="c", subcore_axis_name="s")

    @functools.partial(
        pl.kernel,
        mesh=mesh,
        out_type=jax.ShapeDtypeStruct((n_tokens, D_MODEL), jnp.float32),
        scratch_types=[
            pltpu.VMEM((per_w, _CHUNK), jnp.int32),
            pltpu.VMEM((_CHUNK, D_MODEL), jnp.float32),
            pltpu.SemaphoreType.DMA,
        ],
    )
    def k(table_hbm, idx_hbm, out_hbm, idx_v, rows_v, sem):
        wid = lax.axis_index("s") * _NC + lax.axis_index("c")
        row0 = wid * per_w
        pltpu.sync_copy(idx_hbm.at[wid], idx_v)

        def body(j, carry):
            pltpu.async_copy(table_hbm.at[idx_v.at[j]], rows_v, sem).wait()
            off = pl.multiple_of((row0 + j) * _CHUNK, _CHUNK)
            pltpu.sync_copy(rows_v, out_hbm.at[pl.ds(off, _CHUNK)])
            return carry

        lax.fori_loop(0, per_w, body, 0)

    return k(table_t, idx3d)


def kernel(x, W_E):
    b, p = x.shape
    n = b * p
    table_t = _transpose(W_E)
    idx3d = x.astype(jnp.int32).reshape(_NW, n // (_NW * _CHUNK), _CHUNK)
    out = _gather(table_t, idx3d, n)
    return out.reshape(b, p, D_MODEL)


# speedup vs baseline: 3.6516x; 1.5995x over previous
import functools

import jax
import jax.numpy as jnp
from jax import lax
from jax.experimental import pallas as pl
from jax.experimental.pallas import tpu as pltpu
from jax.experimental.pallas import tpu_sc as plsc

D_MODEL = 128
VOCAB = 100000
_VB = 2048

_NC = 2
_NS = 16
_NW = _NC * _NS
_GB = 8


def _transpose_body(w_ref, o_ref):
    o_ref[...] = w_ref[...].T


def _transpose(W_E):
    return pl.pallas_call(
        _transpose_body,
        grid=((VOCAB + _VB - 1) // _VB,),
        in_specs=[pl.BlockSpec((D_MODEL, _VB), lambda i: (0, i))],
        out_specs=pl.BlockSpec((_VB, D_MODEL), lambda i: (i, 0)),
        out_shape=jax.ShapeDtypeStruct((VOCAB, D_MODEL), jnp.float32),
    )(W_E)


def _gather(table_t, idx3d, batch, n_ctx):
    per_w = idx3d.shape[1]
    n_groups = per_w // _GB
    mesh = plsc.VectorSubcoreMesh(core_axis_name="c", subcore_axis_name="s")

    @functools.partial(
        pl.kernel,
        mesh=mesh,
        out_type=jax.ShapeDtypeStruct((batch, n_ctx, D_MODEL), jnp.float32),
        scratch_types=[
            pltpu.VMEM((per_w, n_ctx), jnp.int32),
            pltpu.VMEM((_GB, n_ctx, D_MODEL), jnp.float32),
            pltpu.VMEM((_GB, n_ctx, D_MODEL), jnp.float32),
            pltpu.SemaphoreType.DMA,
            pltpu.SemaphoreType.DMA,
            pltpu.SemaphoreType.DMA,
        ],
    )
    def k(table_hbm, idx_hbm, out_hbm, idx_v, rows_a, rows_b, gsem, wsem_a,
          wsem_b):
        wid = lax.axis_index("s") * _NC + lax.axis_index("c")
        b0 = wid * per_w
        pltpu.sync_copy(idx_hbm.at[wid], idx_v)

        def do_group(g, rows_v, wsem):
            handles = [
                pltpu.async_copy(
                    table_hbm.at[idx_v.at[g * _GB + i]], rows_v.at[i], gsem)
                for i in range(_GB)
            ]
            for h in handles:
                h.wait()
            pltpu.async_copy(
                rows_v, out_hbm.at[pl.ds(b0 + g * _GB, _GB)], wsem)

        def drain_write(rows_v, wsem):
            pltpu.make_async_copy(
                rows_v, out_hbm.at[pl.ds(b0, _GB)], wsem).wait()

        def body(g, carry):
            even = g % 2 == 0

            @pl.when(jnp.logical_and(g >= 2, even))
            def _():
                drain_write(rows_a, wsem_a)

            @pl.when(jnp.logical_and(g >= 2, jnp.logical_not(even)))
            def _():
                drain_write(rows_b, wsem_b)

            @pl.when(even)
            def _():
                do_group(g, rows_a, wsem_a)

            @pl.when(jnp.logical_not(even))
            def _():
                do_group(g, rows_b, wsem_b)

            return carry

        lax.fori_loop(0, n_groups, body, 0)
        drain_write(rows_a, wsem_a)
        drain_write(rows_b, wsem_b)

    return k(table_t, idx3d)


def kernel(x, W_E):
    b, p = x.shape
    table_t = _transpose(W_E)
    idx3d = x.astype(jnp.int32).reshape(_NW, b // _NW, p)
    return _gather(table_t, idx3d, b, p)


# speedup vs baseline: 3.9659x; 1.0861x over previous
import functools

import jax
import jax.numpy as jnp
from jax import lax
from jax.experimental import pallas as pl
from jax.experimental.pallas import tpu as pltpu
from jax.experimental.pallas import tpu_sc as plsc

D_MODEL = 128
VOCAB = 100000
_VB = 8192

_NC = 2
_NS = 16
_NW = _NC * _NS
_GB = 8


def _transpose_body(w_ref, o_ref):
    o_ref[...] = w_ref[...].T


def _transpose(W_E):
    return pl.pallas_call(
        _transpose_body,
        grid=((VOCAB + _VB - 1) // _VB,),
        in_specs=[pl.BlockSpec((D_MODEL, _VB), lambda i: (0, i))],
        out_specs=pl.BlockSpec((_VB, D_MODEL), lambda i: (i, 0)),
        out_shape=jax.ShapeDtypeStruct((VOCAB, D_MODEL), jnp.float32),
    )(W_E)


def _gather(table_t, idx3d, batch, n_ctx):
    per_w = idx3d.shape[1]
    n_groups = per_w // _GB
    mesh = plsc.VectorSubcoreMesh(core_axis_name="c", subcore_axis_name="s")

    @functools.partial(
        pl.kernel,
        mesh=mesh,
        out_type=jax.ShapeDtypeStruct((batch, n_ctx, D_MODEL), jnp.float32),
        scratch_types=[
            pltpu.VMEM((per_w, n_ctx), jnp.int32),
            pltpu.VMEM((_GB, n_ctx, D_MODEL), jnp.float32),
            pltpu.VMEM((_GB, n_ctx, D_MODEL), jnp.float32),
            pltpu.SemaphoreType.DMA,
            pltpu.SemaphoreType.DMA,
            pltpu.SemaphoreType.DMA,
        ],
    )
    def k(table_hbm, idx_hbm, out_hbm, idx_v, rows_a, rows_b, gsem, wsem_a,
          wsem_b):
        wid = lax.axis_index("s") * _NC + lax.axis_index("c")
        b0 = wid * per_w
        pltpu.sync_copy(idx_hbm.at[wid], idx_v)

        def do_group(g, rows_v, wsem):
            handles = [
                pltpu.async_copy(
                    table_hbm.at[idx_v.at[g * _GB + i]], rows_v.at[i], gsem)
                for i in range(_GB)
            ]
            for h in handles:
                h.wait()
            pltpu.async_copy(
                rows_v, out_hbm.at[pl.ds(b0 + g * _GB, _GB)], wsem)

        def drain_write(rows_v, wsem):
            pltpu.make_async_copy(
                rows_v, out_hbm.at[pl.ds(b0, _GB)], wsem).wait()

        def body(g, carry):
            even = g % 2 == 0

            @pl.when(jnp.logical_and(g >= 2, even))
            def _():
                drain_write(rows_a, wsem_a)

            @pl.when(jnp.logical_and(g >= 2, jnp.logical_not(even)))
            def _():
                drain_write(rows_b, wsem_b)

            @pl.when(even)
            def _():
                do_group(g, rows_a, wsem_a)

            @pl.when(jnp.logical_not(even))
            def _():
                do_group(g, rows_b, wsem_b)

            return carry

        lax.fori_loop(0, n_groups, body, 0)
        drain_write(rows_a, wsem_a)
        drain_write(rows_b, wsem_b)

    return k(table_t, idx3d)


def kernel(x, W_E):
    b, p = x.shape
    table_t = _transpose(W_E)
    idx3d = x.astype(jnp.int32).reshape(_NW, b // _NW, p)
    return _gather(table_t, idx3d, b, p)


# speedup vs baseline: 3.9758x; 1.0025x over previous
import functools

import jax
import jax.numpy as jnp
from jax import lax
from jax.experimental import pallas as pl
from jax.experimental.pallas import tpu as pltpu
from jax.experimental.pallas import tpu_sc as plsc

D_MODEL = 128
VOCAB = 100000
_VB = 16384

_NC = 2
_NS = 16
_NW = _NC * _NS
_GB = 8


def _transpose_body(w_ref, o_ref):
    o_ref[...] = w_ref[...].T


def _transpose(W_E):
    return pl.pallas_call(
        _transpose_body,
        grid=((VOCAB + _VB - 1) // _VB,),
        in_specs=[pl.BlockSpec((D_MODEL, _VB), lambda i: (0, i))],
        out_specs=pl.BlockSpec((_VB, D_MODEL), lambda i: (i, 0)),
        out_shape=jax.ShapeDtypeStruct((VOCAB, D_MODEL), jnp.float32),
    )(W_E)


def _gather(table_t, idx3d, batch, n_ctx):
    per_w = idx3d.shape[1]
    n_groups = per_w // _GB
    mesh = plsc.VectorSubcoreMesh(core_axis_name="c", subcore_axis_name="s")

    @functools.partial(
        pl.kernel,
        mesh=mesh,
        out_type=jax.ShapeDtypeStruct((batch, n_ctx, D_MODEL), jnp.float32),
        scratch_types=[
            pltpu.VMEM((per_w, n_ctx), jnp.int32),
            pltpu.VMEM((_GB, n_ctx, D_MODEL), jnp.float32),
            pltpu.VMEM((_GB, n_ctx, D_MODEL), jnp.float32),
            pltpu.SemaphoreType.DMA,
            pltpu.SemaphoreType.DMA,
            pltpu.SemaphoreType.DMA,
        ],
    )
    def k(table_hbm, idx_hbm, out_hbm, idx_v, rows_a, rows_b, gsem, wsem_a,
          wsem_b):
        wid = lax.axis_index("s") * _NC + lax.axis_index("c")
        b0 = wid * per_w
        pltpu.sync_copy(idx_hbm.at[wid], idx_v)

        def do_group(g, rows_v, wsem):
            handles = [
                pltpu.async_copy(
                    table_hbm.at[idx_v.at[g * _GB + i]], rows_v.at[i], gsem)
                for i in range(_GB)
            ]
            for h in handles:
                h.wait()
            pltpu.async_copy(
                rows_v, out_hbm.at[pl.ds(b0 + g * _GB, _GB)], wsem)

        def drain_write(rows_v, wsem):
            pltpu.make_async_copy(
                rows_v, out_hbm.at[pl.ds(b0, _GB)], wsem).wait()

        def body(g, carry):
            even = g % 2 == 0

            @pl.when(jnp.logical_and(g >= 2, even))
            def _():
                drain_write(rows_a, wsem_a)

            @pl.when(jnp.logical_and(g >= 2, jnp.logical_not(even)))
            def _():
                drain_write(rows_b, wsem_b)

            @pl.when(even)
            def _():
                do_group(g, rows_a, wsem_a)

            @pl.when(jnp.logical_not(even))
            def _():
                do_group(g, rows_b, wsem_b)

            return carry

        lax.fori_loop(0, n_groups, body, 0)
        drain_write(rows_a, wsem_a)
        drain_write(rows_b, wsem_b)

    return k(table_t, idx3d)


def kernel(x, W_E):
    b, p = x.shape
    table_t = _transpose(W_E)
    idx3d = x.astype(jnp.int32).reshape(_NW, b // _NW, p)
    return _gather(table_t, idx3d, b, p)


# speedup vs baseline: 3.9862x; 1.0026x over previous
import functools

import jax
import jax.numpy as jnp
from jax import lax
from jax.experimental import pallas as pl
from jax.experimental.pallas import tpu as pltpu
from jax.experimental.pallas import tpu_sc as plsc

D_MODEL = 128
VOCAB = 100000
_VB = 16384

_NC = 2
_NS = 16
_NW = _NC * _NS
_GB = 8


def _transpose_body(w_ref, o_ref):
    eye = jnp.equal(
        lax.broadcasted_iota(jnp.int32, (D_MODEL, D_MODEL), 0),
        lax.broadcasted_iota(jnp.int32, (D_MODEL, D_MODEL), 1),
    ).astype(jnp.float32)
    o_ref[...] = lax.dot_general(
        w_ref[...], eye, (((0,), (0,)), ((), ())),
        preferred_element_type=jnp.float32)


def _transpose(W_E):
    return pl.pallas_call(
        _transpose_body,
        grid=((VOCAB + _VB - 1) // _VB,),
        in_specs=[pl.BlockSpec((D_MODEL, _VB), lambda i: (0, i))],
        out_specs=pl.BlockSpec((_VB, D_MODEL), lambda i: (i, 0)),
        out_shape=jax.ShapeDtypeStruct((VOCAB, D_MODEL), jnp.float32),
    )(W_E)


def _gather(table_t, idx3d, batch, n_ctx):
    per_w = idx3d.shape[1]
    n_groups = per_w // _GB
    mesh = plsc.VectorSubcoreMesh(core_axis_name="c", subcore_axis_name="s")

    @functools.partial(
        pl.kernel,
        mesh=mesh,
        out_type=jax.ShapeDtypeStruct((batch, n_ctx, D_MODEL), jnp.float32),
        scratch_types=[
            pltpu.VMEM((per_w, n_ctx), jnp.int32),
            pltpu.VMEM((_GB, n_ctx, D_MODEL), jnp.float32),
            pltpu.VMEM((_GB, n_ctx, D_MODEL), jnp.float32),
            pltpu.SemaphoreType.DMA,
            pltpu.SemaphoreType.DMA,
            pltpu.SemaphoreType.DMA,
        ],
    )
    def k(table_hbm, idx_hbm, out_hbm, idx_v, rows_a, rows_b, gsem, wsem_a,
          wsem_b):
        wid = lax.axis_index("s") * _NC + lax.axis_index("c")
        b0 = wid * per_w
        pltpu.sync_copy(idx_hbm.at[wid], idx_v)

        def do_group(g, rows_v, wsem):
            handles = [
                pltpu.async_copy(
                    table_hbm.at[idx_v.at[g * _GB + i]], rows_v.at[i], gsem)
                for i in range(_GB)
            ]
            for h in handles:
                h.wait()
            pltpu.async_copy(
                rows_v, out_hbm.at[pl.ds(b0 + g * _GB, _GB)], wsem)

        def drain_write(rows_v, wsem):
            pltpu.make_async_copy(
                rows_v, out_hbm.at[pl.ds(b0, _GB)], wsem).wait()

        def body(g, carry):
            even = g % 2 == 0

            @pl.when(jnp.logical_and(g >= 2, even))
            def _():
                drain_write(rows_a, wsem_a)

            @pl.when(jnp.logical_and(g >= 2, jnp.logical_not(even)))
            def _():
                drain_write(rows_b, wsem_b)

            @pl.when(even)
            def _():
                do_group(g, rows_a, wsem_a)

            @pl.when(jnp.logical_not(even))
            def _():
                do_group(g, rows_b, wsem_b)

            return carry

        lax.fori_loop(0, n_groups, body, 0)
        drain_write(rows_a, wsem_a)
        drain_write(rows_b, wsem_b)

    return k(table_t, idx3d)


def kernel(x, W_E):
    b, p = x.shape
    table_t = _transpose(W_E)
    idx3d = x.astype(jnp.int32).reshape(_NW, b // _NW, p)
    return _gather(table_t, idx3d, b, p)
